# baseline (speedup 1.0000x reference)
import jax
import jax.numpy as jnp
from jax import lax
from jax.experimental import pallas as pl
from jax.experimental.pallas import tpu as pltpu
from jax.experimental.pallas import tpu_sc as plsc

_B = 8
_C = 192
_G = 96
_HW = 224 * 224
_NW = 32
_RPW = (_B * _G) // _NW
_PAD = 48
_WINDOW = 8


def _sc_body(x_hbm, src_hbm, o1_hbm, o2_hbm, src_v, sem):
    wid = lax.axis_index("s") * 2 + lax.axis_index("c")
    base = wid * _RPW
    pltpu.sync_copy(src_hbm, src_v)

    def drain_one():
        pltpu.make_async_copy(x_hbm.at[0], o1_hbm.at[0], sem).wait()

    issued = 0
    drained = 0
    for o, o_hbm in enumerate((o1_hbm, o2_hbm)):
        for k in range(0, _RPW, 16):
            nval = min(16, _RPW - k)
            v = src_v[pl.ds(o * _NW * _PAD + wid * _PAD + k, 16)]
            for lane in range(nval):
                src = v[lane]
                dst = base + k + lane
                pltpu.async_copy(x_hbm.at[src], o_hbm.at[dst], sem)
                issued += 1
                if issued - drained > _WINDOW:
                    drain_one()
                    drained += 1
    for _ in range(issued - drained):
        drain_one()


def kernel(x, fp_index1, fp_index2):
    b, c, h, w = x.shape
    g = fp_index1.shape[0]
    hw = h * w
    xr = x.reshape(b * c, hw)

    idx = jnp.stack([fp_index1.astype(jnp.int32), fp_index2.astype(jnp.int32)])
    rows = jnp.arange(b * g, dtype=jnp.int32)
    srcs = rows[None, :] // g * c + idx[:, rows % g]
    srcs = srcs.reshape(2, _NW, _RPW)
    srcs = jnp.pad(srcs, ((0, 0), (0, 0), (0, _PAD - _RPW)))
    src_tab = srcs.reshape(-1)

    mesh = plsc.VectorSubcoreMesh(core_axis_name="c", subcore_axis_name="s")
    sc_kernel = pl.kernel(
        _sc_body,
        out_type=[
            jax.ShapeDtypeStruct((b * g, hw), x.dtype),
            jax.ShapeDtypeStruct((b * g, hw), x.dtype),
        ],
        mesh=mesh,
        scratch_types=[
            pltpu.VMEM((2 * _NW * _PAD,), jnp.int32),
            pltpu.SemaphoreType.DMA,
        ],
    )
    out1, out2 = sc_kernel(xr, src_tab)
    return out1.reshape(b, g, h, w), out2.reshape(b, g, h, w)
